# Initial kernel scaffold; baseline (speedup 1.0000x reference)
import jax
import jax.numpy as jnp
from jax.experimental import pallas as pl


def kernel(x, edge_index, W_lin, b_lin, W1, b1, W2, b2, Wih, Whh, bih, bhh, cell_hidden):
    raise NotImplementedError("write your pallas kernel here")


# speedup vs baseline: 10.6235x; 10.6235x over previous
import functools

import jax
import jax.numpy as jnp
from jax import lax
from jax.experimental import pallas as pl
from jax.experimental.pallas import tpu as pltpu
from jax.experimental.pallas import tpu_sc as plsc

NN = 10000
DD = 128
EE = 320000
NC = 2
NS = 16
NW = NC * NS
EPT = EE // NW
CH = 128
NCHUNK = -(-EPT // CH)
EPAD = NCHUNK * CH
NP = 10240
RPT = NP // NS

_mesh = plsc.VectorSubcoreMesh(
    core_axis_name="c", subcore_axis_name="s", num_cores=NC, num_subcores=NS
)
_sc_params = pltpu.CompilerParams(needs_layout_passes=False)


def _zero_rows(ref, nrows):
    z = jnp.zeros((16,), jnp.float32)

    def body(i, _):
        for k in range(8):
            ref[i, pl.ds(k * 16, 16)] = z
        return 0

    lax.fori_loop(0, nrows, body, 0)


def _deg_body(dst_hbm, deg_hbm, dst_v, part_v, tmp_v, obuf_v, sh):
    c = lax.axis_index("c")
    s = lax.axis_index("s")
    w = c * NS + s
    pltpu.sync_copy(dst_hbm.at[w], dst_v)

    z = jnp.zeros((16,), jnp.float32)

    def zbody(i, _):
        part_v[pl.ds(i * 16, 16)] = z
        return 0

    lax.fori_loop(0, NP // 16, zbody, 0)

    ones = jnp.ones((16,), jnp.float32)

    def body(t, _):
        d = dst_v[pl.ds(t * 16, 16)]
        plsc.addupdate_scatter(part_v, [d], ones)
        return 0

    lax.fori_loop(0, EPAD // 16, body, 0)
    pltpu.sync_copy(part_v, sh.at[s])
    plsc.subcore_barrier()
    for p in range(NS):
        pltpu.sync_copy(sh.at[p, pl.ds(s * RPT, RPT)], tmp_v.at[p])

    def rbody(k, _):
        acc = tmp_v[0, pl.ds(k * 16, 16)]
        for p in range(1, NS):
            acc = acc + tmp_v[p, pl.ds(k * 16, 16)]
        obuf_v[pl.ds(k * 16, 16)] = acc
        return 0

    lax.fori_loop(0, RPT // 16, rbody, 0)
    pltpu.sync_copy(obuf_v, deg_hbm.at[c, pl.ds(s * RPT, RPT)])


_deg_call = pl.kernel(
    _deg_body,
    out_type=jax.ShapeDtypeStruct((NC, NP), jnp.float32),
    mesh=_mesh,
    scratch_types=[
        pltpu.VMEM((EPAD,), jnp.int32),
        pltpu.VMEM((NP,), jnp.float32),
        pltpu.VMEM((NS, RPT), jnp.float32),
        pltpu.VMEM((RPT,), jnp.float32),
        pltpu.VMEM_SHARED((NS, NP), jnp.float32),
    ],
    compiler_params=_sc_params,
)


def _agg_body(g_hbm, src_hbm, dst_hbm, out_hbm,
              src_v, dst_v, rows0_v, sem0, acc_sh):
    c = lax.axis_index("c")
    s = lax.axis_index("s")
    w = c * NS + s
    pltpu.sync_copy(src_hbm.at[w], src_v)
    pltpu.sync_copy(dst_hbm.at[w], dst_v)
    _zero_rows(rows0_v, CH)
    for t in range(RPT // CH):
        pltpu.sync_copy(rows0_v, acc_sh.at[pl.ds(s * RPT + t * CH, CH)])
    plsc.subcore_barrier()

    def body(j, _):
        cpa = pltpu.async_copy(g_hbm.at[src_v.at[j]], rows0_v, sem0)
        cpa.wait()
        pltpu.sync_copy(rows0_v, acc_sh.at[dst_v.at[j]], add=True)
        return 0

    lax.fori_loop(0, NCHUNK, body, 0)

    plsc.subcore_barrier()
    for t in range(RPT // CH):
        r = s * RPT + t * CH
        pltpu.sync_copy(acc_sh.at[pl.ds(r, CH)], rows0_v)
        pltpu.sync_copy(rows0_v, out_hbm.at[c, pl.ds(r, CH)])


_agg_call = pl.kernel(
    _agg_body,
    out_type=jax.ShapeDtypeStruct((NC, NP, DD), jnp.float32),
    mesh=_mesh,
    scratch_types=[
        pltpu.VMEM((NCHUNK, CH), jnp.int32),
        pltpu.VMEM((NCHUNK, CH), jnp.int32),
        pltpu.VMEM((CH, DD), jnp.float32),
        pltpu.SemaphoreType.DMA,
        pltpu.VMEM_SHARED((NP, DD), jnp.float32),
    ],
    compiler_params=_sc_params,
)


_RB = 1000
_GRID = NN // _RB

_row_spec = pl.BlockSpec((_RB, DD), lambda i: (i, 0))
_col_spec = pl.BlockSpec((_RB, 1), lambda i: (i, 0))
_w_spec = pl.BlockSpec((DD, DD), lambda i: (0, 0))
_b_spec = pl.BlockSpec((1, DD), lambda i: (0, 0))
_p_spec = pl.BlockSpec((NC, _RB, DD), lambda i: (0, i, 0))
_deg_spec = pl.BlockSpec((NC, _RB, 1), lambda i: (0, i, 0))

_PREC = jax.lax.Precision.HIGHEST


def _mm2_body(x_ref, wl_ref, bl_ref, w1_ref, b1_ref, o_ref):
    h0 = jnp.dot(x_ref[...], wl_ref[...], precision=_PREC,
                 preferred_element_type=jnp.float32) + bl_ref[...]
    o_ref[...] = jnp.dot(h0, w1_ref[...], precision=_PREC,
                         preferred_element_type=jnp.float32) + b1_ref[...]


_mm2 = pl.pallas_call(
    _mm2_body,
    grid=(_GRID,),
    in_specs=[_row_spec, _w_spec, _b_spec, _w_spec, _b_spec],
    out_specs=_row_spec,
    out_shape=jax.ShapeDtypeStruct((NN, DD), jnp.float32),
)


def _scale_body(deg_ref, t1_ref, rs_ref, g1_ref):
    rs = lax.rsqrt(deg_ref[0] + deg_ref[1] + 1.0)
    rs_ref[...] = rs
    g1_ref[...] = t1_ref[...] * rs


_scale = pl.pallas_call(
    _scale_body,
    grid=(_GRID,),
    in_specs=[_deg_spec, _row_spec],
    out_specs=[_col_spec, _row_spec],
    out_shape=[jax.ShapeDtypeStruct((NN, 1), jnp.float32),
               jax.ShapeDtypeStruct((NN, DD), jnp.float32)],
)


def _layer2_body(p_ref, rs_ref, t1_ref, w2_ref, b2_ref, t2_ref, g2_ref):
    rs = rs_ref[...]
    h1 = jnp.maximum(rs * (p_ref[0] + p_ref[1]) + t1_ref[...], 0.0)
    t2 = jnp.dot(h1, w2_ref[...], precision=_PREC,
                 preferred_element_type=jnp.float32) + b2_ref[...]
    t2_ref[...] = t2
    g2_ref[...] = t2 * rs


_layer2 = pl.pallas_call(
    _layer2_body,
    grid=(_GRID,),
    in_specs=[_p_spec, _col_spec, _row_spec, _w_spec, _b_spec],
    out_specs=[_row_spec, _row_spec],
    out_shape=[jax.ShapeDtypeStruct((NN, DD), jnp.float32),
               jax.ShapeDtypeStruct((NN, DD), jnp.float32)],
)


def _final_body(q_ref, rs_ref, t2_ref, wih_ref, whh_ref, b_ref, ch_ref, o_ref):
    h2 = jnp.maximum(rs_ref[...] * (q_ref[0] + q_ref[1]) + t2_ref[...], 0.0)
    o_ref[...] = jnp.tanh(
        jnp.dot(h2, wih_ref[...], precision=_PREC,
                preferred_element_type=jnp.float32)
        + jnp.dot(ch_ref[...], whh_ref[...], precision=_PREC,
                  preferred_element_type=jnp.float32)
        + b_ref[...])


_final = pl.pallas_call(
    _final_body,
    grid=(_GRID,),
    in_specs=[_p_spec, _col_spec, _row_spec, _w_spec, _w_spec, _b_spec,
              _row_spec],
    out_specs=_row_spec,
    out_shape=jax.ShapeDtypeStruct((NN, DD), jnp.float32),
)


def kernel(x, edge_index, W_lin, b_lin, W1, b1, W2, b2, Wih, Whh, bih, bhh,
           cell_hidden):
    ei = edge_index.astype(jnp.int32)
    src = ei[0].reshape(NW, EPT)
    dst = ei[1].reshape(NW, EPT)
    pad = EPAD - EPT
    srcp = jnp.pad(src, ((0, 0), (0, pad))).reshape(NW, NCHUNK, CH)
    dstp = jnp.pad(dst, ((0, 0), (0, pad)),
                   constant_values=NN).reshape(NW, NCHUNK, CH)

    degp = _deg_call(dstp.reshape(NW, EPAD))
    t1 = _mm2(x, W_lin, b_lin.reshape(1, DD), W1, b1.reshape(1, DD))
    degn = degp.reshape(NC, NP, 1)
    rs, g1 = _scale(degn, t1)
    p = _agg_call(g1, srcp, dstp)
    t2, g2 = _layer2(p, rs, t1, W2, b2.reshape(1, DD))
    q = _agg_call(g2, srcp, dstp)
    out = _final(q, rs, t2, Wih.T, Whh.T,
                 (bih + bhh).reshape(1, DD), cell_hidden)
    return out
